# baseline (speedup 1.0000x reference)
import functools

import jax
import jax.numpy as jnp
from jax import lax
from jax.experimental import pallas as pl
from jax.experimental.pallas import tpu as pltpu
from jax.experimental.pallas import tpu_sc as plsc

IN_F = 768
OUT_F = 768
H = 64
E = 64
NROUTE = 4
SSF = 0.95
N = 8192
B = 256
NBLK = 96
PB = NBLK * B
NW = 32
JT = N // NW

f32 = jnp.float32
i32 = jnp.int32

_SCALES = [1.0]
for _ in range(NROUTE):
    _SCALES.append(_SCALES[-1] * SSF)


def _route(logits):
    m = jnp.max(logits, axis=-1, keepdims=True)
    s = jnp.sum(jnp.exp(logits - m), axis=-1, keepdims=True)
    topv = 1.0 / s
    gate = topv / (topv + 1e-9)
    lane_e = lax.broadcasted_iota(i32, logits.shape, 1)
    idx = jnp.min(jnp.where(logits == m, lane_e, E), axis=-1, keepdims=True)
    return idx, gate


def _r0_body(x_ref, win_ref, bin_ref, wr_ref, br_ref,
             hs_ref, eid_ref, gate_ref):
    h = jnp.maximum(
        jnp.dot(x_ref[...], win_ref[...], preferred_element_type=f32)
        + bin_ref[...], 0.0)
    logits = jnp.dot(h, wr_ref[...], preferred_element_type=f32) + br_ref[...]
    idx, gate = _route(logits)
    hs_ref[...] = h
    eid_ref[...] = idx
    gate_ref[...] = gate


def _exp_route_body(be_ref, h_ref, g_ref, w_ref, b_ref, wr_ref, br_ref,
                    hs_ref, eid_ref, gate_ref, *, scale):
    hn = jnp.maximum(
        jnp.dot(h_ref[...], w_ref[0], preferred_element_type=f32)
        + b_ref[0, 0], 0.0)
    hm = (g_ref[...] * hn) * scale
    logits = jnp.dot(hm, wr_ref[...], preferred_element_type=f32) + br_ref[...]
    idx, gate = _route(logits)
    hs_ref[...] = hm
    eid_ref[...] = idx
    gate_ref[...] = gate


def _exp_final_body(be_ref, h_ref, g_ref, w_ref, b_ref, out_ref, *, scale):
    hn = jnp.maximum(
        jnp.dot(h_ref[...], w_ref[0], preferred_element_type=f32)
        + b_ref[0, 0], 0.0)
    out_ref[...] = (g_ref[...] * hn) * scale


def _out_body(h_ref, wout_ref, bout_ref, o_ref):
    o_ref[...] = jnp.dot(h_ref[...], wout_ref[...],
                         preferred_element_type=f32) + bout_ref[...]


_r0_call = pl.pallas_call(
    _r0_body,
    grid=(N // B,),
    in_specs=[
        pl.BlockSpec((B, IN_F), lambda i: (i, 0)),
        pl.BlockSpec((IN_F, H), lambda i: (0, 0)),
        pl.BlockSpec((H,), lambda i: (0,)),
        pl.BlockSpec((H, E), lambda i: (0, 0)),
        pl.BlockSpec((E,), lambda i: (0,)),
    ],
    out_specs=[pl.BlockSpec((B, H), lambda i: (i, 0)),
               pl.BlockSpec((B, 1), lambda i: (i, 0)),
               pl.BlockSpec((B, 1), lambda i: (i, 0))],
    out_shape=[jax.ShapeDtypeStruct((N, H), f32),
               jax.ShapeDtypeStruct((N, 1), i32),
               jax.ShapeDtypeStruct((N, 1), f32)],
)


def _make_exp(step):
    last = step == NROUTE - 1
    scale = _SCALES[step]
    if last:
        body = functools.partial(_exp_final_body, scale=scale)
        in_specs = [
            pl.BlockSpec((B, H), lambda i, be: (i, 0)),
            pl.BlockSpec((B, 1), lambda i, be: (i, 0)),
            pl.BlockSpec((1, H, H), lambda i, be: (be[i], 0, 0)),
            pl.BlockSpec((1, 1, H), lambda i, be: (be[i], 0, 0)),
        ]
        out_specs = pl.BlockSpec((B, H), lambda i, be: (i, 0))
        out_shape = jax.ShapeDtypeStruct((PB, H), f32)
    else:
        body = functools.partial(_exp_route_body, scale=scale)
        in_specs = [
            pl.BlockSpec((B, H), lambda i, be: (i, 0)),
            pl.BlockSpec((B, 1), lambda i, be: (i, 0)),
            pl.BlockSpec((1, H, H), lambda i, be: (be[i], 0, 0)),
            pl.BlockSpec((1, 1, H), lambda i, be: (be[i], 0, 0)),
            pl.BlockSpec((H, E), lambda i, be: (0, 0)),
            pl.BlockSpec((E,), lambda i, be: (0,)),
        ]
        out_specs = [pl.BlockSpec((B, H), lambda i, be: (i, 0)),
                     pl.BlockSpec((B, 1), lambda i, be: (i, 0)),
                     pl.BlockSpec((B, 1), lambda i, be: (i, 0))]
        out_shape = [jax.ShapeDtypeStruct((PB, H), f32),
                     jax.ShapeDtypeStruct((PB, 1), i32),
                     jax.ShapeDtypeStruct((PB, 1), f32)]
    return pl.pallas_call(
        body,
        grid_spec=pltpu.PrefetchScalarGridSpec(
            num_scalar_prefetch=1,
            grid=(NBLK,),
            in_specs=in_specs,
            out_specs=out_specs,
        ),
        out_shape=out_shape,
    )


_exp_calls = [_make_exp(t) for t in range(NROUTE)]

_out_call = pl.pallas_call(
    _out_body,
    grid=(N // B,),
    in_specs=[
        pl.BlockSpec((B, H), lambda i: (i, 0)),
        pl.BlockSpec((H, OUT_F), lambda i: (0, 0)),
        pl.BlockSpec((OUT_F,), lambda i: (0,)),
    ],
    out_specs=pl.BlockSpec((B, OUT_F), lambda i: (i, 0)),
    out_shape=jax.ShapeDtypeStruct((N, OUT_F), f32),
)


_mesh = plsc.VectorSubcoreMesh(core_axis_name="c", subcore_axis_name="s")
_sc_params = pltpu.CompilerParams(use_tc_tiling_on_sc=False, needs_layout_passes=False)

_GDN = lax.GatherDimensionNumbers(
    offset_dims=(), collapsed_slice_dims=(0,), start_index_map=(0,))


def _take16(x, idx):
    return lax.gather(x, idx[:, None], _GDN, (1,),
                      mode=lax.GatherScatterMode.PROMISE_IN_BOUNDS)


def _wid():
    return lax.axis_index("s") * 2 + lax.axis_index("c")


def _sort1_body(eid_h, slot2_h, ec2_h, rank2_h, hist2_h,
                slot_v, eid_v, rank_v, hist_v, sem):
    wid = _wid()
    pltpu.sync_copy(slot2_h.at[pl.ds(wid * 2, 2)], slot_v)
    for cc in range(2):
        pltpu.async_copy(eid_h.at[slot_v.at[cc]], eid_v.at[cc], sem).wait()
    lane = lax.iota(i32, 16)
    for q in range(4):
        hist_v[pl.ds(q * 16, 16)] = jnp.zeros((16,), i32)
    for ch in range(16):
        chi, clo = divmod(ch, 8)
        e16 = eid_v[chi, pl.ds(clo * 16, 16)]
        rback = jnp.zeros((16,), i32)
        rfwd = jnp.zeros((16,), i32)
        for sft in range(1, 16):
            bk = _take16(e16, jnp.maximum(lane - sft, 0))
            rback = rback + ((lane >= sft) & (bk == e16)).astype(i32)
            fw = _take16(e16, jnp.minimum(lane + sft, 15))
            rfwd = rfwd + ((lane <= 15 - sft) & (fw == e16)).astype(i32)
        prev = plsc.load_gather(hist_v, [e16])
        plsc.store_scatter(hist_v, [e16], prev + rback + rfwd + 1)
        rank_v[chi, pl.ds(clo * 16, 16)] = prev + rback
    pltpu.sync_copy(rank_v, rank2_h.at[pl.ds(wid * 2, 2)])
    pltpu.sync_copy(eid_v, ec2_h.at[pl.ds(wid * 2, 2)])
    pltpu.sync_copy(hist_v, hist2_h.at[wid])


def _make_sort1(pbin):
    return pl.kernel(
        _sort1_body,
        out_type=(jax.ShapeDtypeStruct((NW * 2, 128), i32),
                  jax.ShapeDtypeStruct((NW * 2, 128), i32),
                  jax.ShapeDtypeStruct((NW, 64), i32)),
        mesh=_mesh,
        compiler_params=_sc_params,
        scratch_types=[pltpu.VMEM((2, 128), i32),
                       pltpu.VMEM((2, 128), i32),
                       pltpu.VMEM((2, 128), i32),
                       pltpu.VMEM((64,), i32),
                       pltpu.SemaphoreType.DMA],
    )


def _sort2_body(ec2_h, rank2_h, slot2_h, orig2_h, hist2_h, h_h, gate_h,
                hs_h, gateS_h, slotO_h, origO_h, be_h,
                grid_v, cnt_v, off_v, base_v, cbase_v,
                ec_v, rank_v, slot_v, orig_v, pos_v, j2_v,
                hbuf, gv, be_v, sem):
    wid = _wid()
    pltpu.sync_copy(hist2_h, grid_v)
    for q in range(4):
        acc = jnp.zeros((16,), i32)
        pre = jnp.zeros((16,), i32)
        for t in range(NW):
            v = grid_v[t, pl.ds(q * 16, 16)]
            acc = acc + v
            pre = pre + v * (t < wid).astype(i32)
        cnt_v[pl.ds(q * 16, 16)] = acc
        base_v[pl.ds(q * 16, 16)] = pre
    carry = jnp.zeros((), i32)
    ccarry = jnp.zeros((), i32)
    for q in range(4):
        sl = pl.ds(q * 16, 16)
        c16 = cnt_v[sl]
        p16 = ((c16 + (B - 1)) // B) * B
        off16 = plsc.cumsum(p16) - p16 + carry
        coff16 = plsc.cumsum(c16) - c16 + ccarry
        pre16 = base_v[sl]
        off_v[sl] = off16
        base_v[sl] = off16 + pre16
        cbase_v[sl] = coff16 + pre16
        carry = carry + jnp.sum(p16)
        ccarry = ccarry + jnp.sum(c16)
    rows = pl.ds(wid * 2, 2)
    pltpu.sync_copy(ec2_h.at[rows], ec_v)
    pltpu.sync_copy(rank2_h.at[rows], rank_v)
    pltpu.sync_copy(slot2_h.at[rows], slot_v)
    pltpu.sync_copy(orig2_h.at[rows], orig_v)
    for ch in range(16):
        chi, clo = divmod(ch, 8)
        sl = pl.ds(clo * 16, 16)
        e16 = ec_v[chi, sl]
        r16 = rank_v[chi, sl]
        pos_v[chi, sl] = plsc.load_gather(base_v, [e16]) + r16
        j2_v[chi, sl] = plsc.load_gather(cbase_v, [e16]) + r16
    for cc in range(2):
        pltpu.async_copy(h_h.at[slot_v.at[cc]],
                         hbuf.at[pl.ds(cc * 128, 128)], sem).wait()
        pltpu.async_copy(gate_h.at[slot_v.at[cc]], gv.at[cc], sem).wait()
    for cc in range(2):
        pltpu.async_copy(hbuf.at[pl.ds(cc * 128, 128)],
                         hs_h.at[pos_v.at[cc]], sem).wait()
        pltpu.async_copy(gv.at[cc], gateS_h.at[pos_v.at[cc]], sem).wait()
        pltpu.async_copy(pos_v.at[cc], slotO_h.at[j2_v.at[cc]], sem).wait()
        pltpu.async_copy(orig_v.at[cc], origO_h.at[j2_v.at[cc]], sem).wait()

    @pl.when(wid == 0)
    def _():
        for bq in range(8):
            bvals = (lax.iota(i32, 16) + bq * 16) * B
            acc = jnp.zeros((16,), i32)
            for q in range(4):
                o16 = off_v[pl.ds(q * 16, 16)]
                for sft in range(16):
                    ob = _take16(o16, jnp.full((16,), sft, i32))
                    acc = acc + (ob <= bvals).astype(i32)
            be_v[pl.ds(bq * 16, 16)] = acc - 1
        pltpu.sync_copy(be_v, be_h)


def _make_sort2(pbin):
    return pl.kernel(
        _sort2_body,
        out_type=(jax.ShapeDtypeStruct((PB, H), f32),
                  jax.ShapeDtypeStruct((PB,), f32),
                  jax.ShapeDtypeStruct((N,), i32),
                  jax.ShapeDtypeStruct((N,), i32),
                  jax.ShapeDtypeStruct((128,), i32)),
        mesh=_mesh,
        compiler_params=_sc_params,
        scratch_types=[pltpu.VMEM((NW, 64), i32),
                       pltpu.VMEM((64,), i32),
                       pltpu.VMEM((64,), i32),
                       pltpu.VMEM((64,), i32),
                       pltpu.VMEM((64,), i32),
                       pltpu.VMEM((2, 128), i32),
                       pltpu.VMEM((2, 128), i32),
                       pltpu.VMEM((2, 128), i32),
                       pltpu.VMEM((2, 128), i32),
                       pltpu.VMEM((2, 128), i32),
                       pltpu.VMEM((2, 128), i32),
                       pltpu.VMEM((256, H), f32),
                       pltpu.VMEM((2, 128), f32),
                       pltpu.VMEM((128,), i32),
                       pltpu.SemaphoreType.DMA],
    )


def _sf_body(slot2_h, orig2_h, h_h, hf_h, slot_v, orig_v, hbuf, sem):
    wid = _wid()
    rows = pl.ds(wid * 2, 2)
    pltpu.sync_copy(slot2_h.at[rows], slot_v)
    pltpu.sync_copy(orig2_h.at[rows], orig_v)
    for cc in range(2):
        pltpu.async_copy(h_h.at[slot_v.at[cc]],
                         hbuf.at[pl.ds(cc * 128, 128)], sem).wait()
    for cc in range(2):
        pltpu.async_copy(hbuf.at[pl.ds(cc * 128, 128)],
                         hf_h.at[orig_v.at[cc]], sem).wait()


_sf_call = pl.kernel(
    _sf_body,
    out_type=jax.ShapeDtypeStruct((N, H), f32),
    mesh=_mesh,
    compiler_params=_sc_params,
    scratch_types=[pltpu.VMEM((2, 128), i32),
                   pltpu.VMEM((2, 128), i32),
                   pltpu.VMEM((256, H), f32),
                   pltpu.SemaphoreType.DMA],
)

_sort1_calls = {n: _make_sort1(n) for n in (N, PB)}
_sort2_calls = {n: _make_sort2(n) for n in (N, PB)}


@jax.jit
def kernel(x, W_in, b_in, W_router, b_router, W_experts, b_experts, W_out, b_out):
    x = x.reshape(x.shape[0], -1)
    b_exp3 = b_experts.reshape(E, 1, H)
    hs, eid, gate = _r0_call(x, W_in, b_in, W_router, b_router)
    ident = jnp.arange(N, dtype=i32).reshape(NW * 2, 128)
    slot2 = ident
    orig2 = ident
    h_last = None
    for t in range(NROUTE):
        pbin = N if t == 0 else PB
        ec2, rank2, hist2 = _sort1_calls[pbin](eid.reshape(pbin), slot2)
        hs_s, gate_s, slotO, origO, be = _sort2_calls[pbin](
            ec2, rank2, slot2, orig2, hist2, hs, gate.reshape(pbin))
        slot2 = slotO.reshape(NW * 2, 128)
        orig2 = origO.reshape(NW * 2, 128)
        gate_s = gate_s.reshape(PB, 1)
        if t < NROUTE - 1:
            hs, eid, gate = _exp_calls[t](be, hs_s, gate_s, W_experts, b_exp3,
                                          W_router, b_router)
        else:
            h_last = _exp_calls[t](be, hs_s, gate_s, W_experts, b_exp3)
    hf = _sf_call(slot2, orig2, h_last)
    return _out_call(hf, W_out, b_out)


# speedup vs baseline: 1.5919x; 1.5919x over previous
import functools

import jax
import jax.numpy as jnp
from jax import lax
from jax.experimental import pallas as pl
from jax.experimental.pallas import tpu as pltpu
from jax.experimental.pallas import tpu_sc as plsc

IN_F = 768
OUT_F = 768
H = 64
E = 64
NROUTE = 4
SSF = 0.95
N = 8192
B = 256
NBLK = 96
PB = NBLK * B
NW = 32
JT = N // NW
W = 80
GCOL = 64
ECOL = 65

f32 = jnp.float32
i32 = jnp.int32

_SCALES = [1.0]
for _ in range(NROUTE):
    _SCALES.append(_SCALES[-1] * SSF)


def _route(logits):
    m = jnp.max(logits, axis=-1, keepdims=True)
    s = jnp.sum(jnp.exp(logits - m), axis=-1, keepdims=True)
    topv = 1.0 / s
    gate = topv / (topv + 1e-9)
    lane_e = lax.broadcasted_iota(i32, logits.shape, 1)
    idx = jnp.min(jnp.where(logits == m, lane_e, E), axis=-1, keepdims=True)
    return idx, gate


def _r0_body(x_ref, win_ref, bin_ref, wr_ref, br_ref, hx_ref):
    h = jnp.maximum(
        jnp.dot(x_ref[...], win_ref[...], preferred_element_type=f32)
        + bin_ref[...], 0.0)
    logits = jnp.dot(h, wr_ref[...], preferred_element_type=f32) + br_ref[...]
    idx, gate = _route(logits)
    hx_ref[:, 0:H] = h
    hx_ref[:, GCOL:GCOL + 1] = gate
    hx_ref[:, ECOL:ECOL + 1] = idx.astype(f32)
    hx_ref[:, ECOL + 1:W] = jnp.zeros((B, W - ECOL - 1), f32)


def _exp_route_body(be_ref, hx_ref, w_ref, b_ref, wr_ref, br_ref,
                    ho_ref, *, scale):
    h = hx_ref[:, 0:H]
    g = hx_ref[:, GCOL:GCOL + 1]
    hn = jnp.maximum(
        jnp.dot(h, w_ref[0], preferred_element_type=f32) + b_ref[0, 0], 0.0)
    hm = (g * hn) * scale
    logits = jnp.dot(hm, wr_ref[...], preferred_element_type=f32) + br_ref[...]
    idx, gate = _route(logits)
    ho_ref[:, 0:H] = hm
    ho_ref[:, GCOL:GCOL + 1] = gate
    ho_ref[:, ECOL:ECOL + 1] = idx.astype(f32)
    ho_ref[:, ECOL + 1:W] = jnp.zeros((B, W - ECOL - 1), f32)


def _exp_final_body(be_ref, hx_ref, w_ref, b_ref, out_ref, *, scale):
    h = hx_ref[:, 0:H]
    g = hx_ref[:, GCOL:GCOL + 1]
    hn = jnp.maximum(
        jnp.dot(h, w_ref[0], preferred_element_type=f32) + b_ref[0, 0], 0.0)
    out_ref[...] = (g * hn) * scale


def _out_body(h_ref, wout_ref, bout_ref, o_ref):
    o_ref[...] = jnp.dot(h_ref[...], wout_ref[...],
                         preferred_element_type=f32) + bout_ref[...]


_r0_call = pl.pallas_call(
    _r0_body,
    grid=(N // B,),
    in_specs=[
        pl.BlockSpec((B, IN_F), lambda i: (i, 0)),
        pl.BlockSpec((IN_F, H), lambda i: (0, 0)),
        pl.BlockSpec((H,), lambda i: (0,)),
        pl.BlockSpec((H, E), lambda i: (0, 0)),
        pl.BlockSpec((E,), lambda i: (0,)),
    ],
    out_specs=pl.BlockSpec((B, W), lambda i: (i, 0)),
    out_shape=jax.ShapeDtypeStruct((N, W), f32),
)


def _make_exp(step):
    last = step == NROUTE - 1
    scale = _SCALES[step]
    if last:
        body = functools.partial(_exp_final_body, scale=scale)
        in_specs = [
            pl.BlockSpec((B, W), lambda i, be: (i, 0)),
            pl.BlockSpec((1, H, H), lambda i, be: (be[i], 0, 0)),
            pl.BlockSpec((1, 1, H), lambda i, be: (be[i], 0, 0)),
        ]
        out_specs = pl.BlockSpec((B, H), lambda i, be: (i, 0))
        out_shape = jax.ShapeDtypeStruct((PB, H), f32)
    else:
        body = functools.partial(_exp_route_body, scale=scale)
        in_specs = [
            pl.BlockSpec((B, W), lambda i, be: (i, 0)),
            pl.BlockSpec((1, H, H), lambda i, be: (be[i], 0, 0)),
            pl.BlockSpec((1, 1, H), lambda i, be: (be[i], 0, 0)),
            pl.BlockSpec((H, E), lambda i, be: (0, 0)),
            pl.BlockSpec((E,), lambda i, be: (0,)),
        ]
        out_specs = pl.BlockSpec((B, W), lambda i, be: (i, 0))
        out_shape = jax.ShapeDtypeStruct((PB, W), f32)
    return pl.pallas_call(
        body,
        grid_spec=pltpu.PrefetchScalarGridSpec(
            num_scalar_prefetch=1,
            grid=(NBLK,),
            in_specs=in_specs,
            out_specs=out_specs,
        ),
        out_shape=out_shape,
    )


_exp_calls = [_make_exp(t) for t in range(NROUTE)]

_out_call = pl.pallas_call(
    _out_body,
    grid=(N // B,),
    in_specs=[
        pl.BlockSpec((B, H), lambda i: (i, 0)),
        pl.BlockSpec((H, OUT_F), lambda i: (0, 0)),
        pl.BlockSpec((OUT_F,), lambda i: (0,)),
    ],
    out_specs=pl.BlockSpec((B, OUT_F), lambda i: (i, 0)),
    out_shape=jax.ShapeDtypeStruct((N, OUT_F), f32),
)


_mesh = plsc.VectorSubcoreMesh(core_axis_name="c", subcore_axis_name="s")
_sc_params = pltpu.CompilerParams(use_tc_tiling_on_sc=False,
                                  needs_layout_passes=False)

_GDN = lax.GatherDimensionNumbers(
    offset_dims=(), collapsed_slice_dims=(0,), start_index_map=(0,))


def _take16(x, idx):
    return lax.gather(x, idx[:, None], _GDN, (1,),
                      mode=lax.GatherScatterMode.PROMISE_IN_BOUNDS)


def _wid():
    return lax.axis_index("s") * 2 + lax.axis_index("c")


def _extract_eid(hxbuf, ch):
    lane = lax.iota(i32, 16)
    row16 = lane + ch * 16
    col16 = jnp.full((16,), ECOL, i32)
    ef = plsc.load_gather(hxbuf, [row16, col16])
    return ef.astype(i32)


def _sort1_body(slot2_h, hx_h, hxc_h, rank2_h, hist2_h,
                slot_v, hxbuf, rank_v, hist_v, sem):
    wid = _wid()
    pltpu.sync_copy(slot2_h.at[pl.ds(wid * 2, 2)], slot_v)
    for cc in range(2):
        pltpu.async_copy(hx_h.at[slot_v.at[cc]],
                         hxbuf.at[pl.ds(cc * 128, 128)], sem).wait()
    lane = lax.iota(i32, 16)
    for q in range(4):
        hist_v[pl.ds(q * 16, 16)] = jnp.zeros((16,), i32)
    for ch in range(16):
        chi, clo = divmod(ch, 8)
        e16 = _extract_eid(hxbuf, ch)
        rback = jnp.zeros((16,), i32)
        rfwd = jnp.zeros((16,), i32)
        for sft in range(1, 16):
            bk = _take16(e16, jnp.maximum(lane - sft, 0))
            rback = rback + ((lane >= sft) & (bk == e16)).astype(i32)
            fw = _take16(e16, jnp.minimum(lane + sft, 15))
            rfwd = rfwd + ((lane <= 15 - sft) & (fw == e16)).astype(i32)
        prev = plsc.load_gather(hist_v, [e16])
        plsc.store_scatter(hist_v, [e16], prev + rback + rfwd + 1)
        rank_v[chi, pl.ds(clo * 16, 16)] = prev + rback
    pltpu.sync_copy(hxbuf, hxc_h.at[pl.ds(wid * JT, JT)])
    pltpu.sync_copy(rank_v, rank2_h.at[pl.ds(wid * 2, 2)])
    pltpu.sync_copy(hist_v, hist2_h.at[wid])


def _make_sort1(pbin):
    return pl.kernel(
        _sort1_body,
        out_type=(jax.ShapeDtypeStruct((N, W), f32),
                  jax.ShapeDtypeStruct((NW * 2, 128), i32),
                  jax.ShapeDtypeStruct((NW, 64), i32)),
        mesh=_mesh,
        compiler_params=_sc_params,
        scratch_types=[pltpu.VMEM((2, 128), i32),
                       pltpu.VMEM((JT, W), f32),
                       pltpu.VMEM((2, 128), i32),
                       pltpu.VMEM((64,), i32),
                       pltpu.SemaphoreType.DMA],
    )


def _sort2_body(hxc_h, rank2_h, hist2_h,
                hs_h, slotO_h, be_h,
                grid_v, cnt_v, off_v, base_v,
                rank_v, pos_v, hxbuf, be_v, sem):
    wid = _wid()
    pltpu.sync_copy(hist2_h, grid_v)
    pltpu.sync_copy(hxc_h.at[pl.ds(wid * JT, JT)], hxbuf)
    pltpu.sync_copy(rank2_h.at[pl.ds(wid * 2, 2)], rank_v)
    for q in range(4):
        acc = jnp.zeros((16,), i32)
        pre = jnp.zeros((16,), i32)
        for t in range(NW):
            v = grid_v[t, pl.ds(q * 16, 16)]
            acc = acc + v
            pre = pre + v * (t < wid).astype(i32)
        cnt_v[pl.ds(q * 16, 16)] = acc
        base_v[pl.ds(q * 16, 16)] = pre
    carry = jnp.zeros((), i32)
    for q in range(4):
        sl = pl.ds(q * 16, 16)
        c16 = cnt_v[sl]
        p16 = ((c16 + (B - 1)) // B) * B
        off16 = plsc.cumsum(p16) - p16 + carry
        pre16 = base_v[sl]
        off_v[sl] = off16
        base_v[sl] = off16 + pre16
        carry = carry + jnp.sum(p16)
    for ch in range(16):
        chi, clo = divmod(ch, 8)
        e16 = _extract_eid(hxbuf, ch)
        r16 = rank_v[chi, pl.ds(clo * 16, 16)]
        pos_v[chi, pl.ds(clo * 16, 16)] = plsc.load_gather(base_v, [e16]) + r16
    for cc in range(2):
        pltpu.async_copy(hxbuf.at[pl.ds(cc * 128, 128)],
                         hs_h.at[pos_v.at[cc]], sem).wait()
    pltpu.sync_copy(pos_v, slotO_h.at[pl.ds(wid * 2, 2)])

    @pl.when(wid == 0)
    def _():
        for bq in range(8):
            bvals = (lax.iota(i32, 16) + bq * 16) * B
            acc = jnp.zeros((16,), i32)
            for q in range(4):
                o16 = off_v[pl.ds(q * 16, 16)]
                for sft in range(16):
                    ob = _take16(o16, jnp.full((16,), sft, i32))
                    acc = acc + (ob <= bvals).astype(i32)
            be_v[pl.ds(bq * 16, 16)] = acc - 1
        pltpu.sync_copy(be_v, be_h)


_sort2_call = pl.kernel(
    _sort2_body,
    out_type=(jax.ShapeDtypeStruct((PB, W), f32),
              jax.ShapeDtypeStruct((NW * 2, 128), i32),
              jax.ShapeDtypeStruct((128,), i32)),
    mesh=_mesh,
    compiler_params=_sc_params,
    scratch_types=[pltpu.VMEM((NW, 64), i32),
                   pltpu.VMEM((64,), i32),
                   pltpu.VMEM((64,), i32),
                   pltpu.VMEM((64,), i32),
                   pltpu.VMEM((2, 128), i32),
                   pltpu.VMEM((2, 128), i32),
                   pltpu.VMEM((JT, W), f32),
                   pltpu.VMEM((128,), i32),
                   pltpu.SemaphoreType.DMA],
)


def _sf_body(slot2_h, h_h, hf_h, slot_v, hbuf, sem):
    wid = _wid()
    pltpu.sync_copy(slot2_h.at[pl.ds(wid * 2, 2)], slot_v)
    for cc in range(2):
        pltpu.async_copy(h_h.at[slot_v.at[cc]],
                         hbuf.at[pl.ds(cc * 128, 128)], sem).wait()
    pltpu.sync_copy(hbuf, hf_h.at[pl.ds(wid * JT, JT)])


_sf_call = pl.kernel(
    _sf_body,
    out_type=jax.ShapeDtypeStruct((N, H), f32),
    mesh=_mesh,
    compiler_params=_sc_params,
    scratch_types=[pltpu.VMEM((2, 128), i32),
                   pltpu.VMEM((JT, H), f32),
                   pltpu.SemaphoreType.DMA],
)

_sort1_calls = {n: _make_sort1(n) for n in (N, PB)}


@jax.jit
def kernel(x, W_in, b_in, W_router, b_router, W_experts, b_experts, W_out, b_out):
    x = x.reshape(x.shape[0], -1)
    b_exp3 = b_experts.reshape(E, 1, H)
    hx = _r0_call(x, W_in, b_in, W_router, b_router)
    slot2 = jnp.arange(N, dtype=i32).reshape(NW * 2, 128)
    h_last = None
    for t in range(NROUTE):
        pbin = N if t == 0 else PB
        hxc, rank2, hist2 = _sort1_calls[pbin](slot2, hx)
        hx_s, slot2, be = _sort2_call(hxc, rank2, hist2)
        if t < NROUTE - 1:
            hx = _exp_calls[t](be, hx_s, W_experts, b_exp3,
                               W_router, b_router)
        else:
            h_last = _exp_calls[t](be, hx_s, W_experts, b_exp3)
    hf = _sf_call(slot2, h_last)
    return _out_call(hf, W_out, b_out)


# speedup vs baseline: 4.0576x; 2.5490x over previous
import functools

import jax
import jax.numpy as jnp
from jax.experimental import pallas as pl

IN_FEATURES = 768
OUT_FEATURES = 768
HIDDEN = 64
NUM_EXPERTS = 64
MAX_ROUTING = 4
SSF = 0.95
N_TOK = 8192

BT = 256


def _rmlp_block(x_ref, w_in_ref, b_in_ref, w_router_ref, b_router_ref,
                w_flat_ref, b_exp_ref, w_out_ref, b_out_ref, out_ref):
    f32 = jnp.float32
    x = x_ref[...]
    h = jnp.maximum(
        jnp.dot(x, w_in_ref[...], preferred_element_type=f32,
                ) + b_in_ref[...], 0.0)

    lane_e = jax.lax.broadcasted_iota(jnp.int32, (BT, NUM_EXPERTS), 1)
    lane_big = jax.lax.broadcasted_iota(jnp.int32, (BT, NUM_EXPERTS * HIDDEN), 1)
    grp_big = lane_big // HIDDEN

    scale = 1.0
    for _ in range(MAX_ROUTING):
        logits = jnp.dot(h, w_router_ref[...], preferred_element_type=f32,
) + b_router_ref[...]
        m = jnp.max(logits, axis=-1, keepdims=True)
        s = jnp.sum(jnp.exp(logits - m), axis=-1, keepdims=True)
        topv = 1.0 / s
        gate = topv / (topv + 1e-9)
        idx = jnp.min(jnp.where(logits == m, lane_e, NUM_EXPERTS),
                      axis=-1, keepdims=True)
        onehot = (lane_e == idx).astype(f32)
        big = jnp.tile(h, (1, NUM_EXPERTS))
        M = jnp.where(grp_big == idx, big, 0.0)
        b_sel = jnp.dot(onehot, b_exp_ref[...], preferred_element_type=f32,
)
        eo = jnp.maximum(
            jnp.dot(M, w_flat_ref[...], preferred_element_type=f32,
) + b_sel, 0.0)
        h = (gate * eo) * scale
        scale = scale * SSF

    out_ref[...] = jnp.dot(h, w_out_ref[...], preferred_element_type=f32) \
        + b_out_ref[...]


@jax.jit
def kernel(x, W_in, b_in, W_router, b_router, W_experts, b_experts, W_out, b_out):
    x = x.reshape(x.shape[0], -1)
    n = x.shape[0]
    w_flat = W_experts.reshape(NUM_EXPERTS * HIDDEN, HIDDEN)

    full = lambda shape: pl.BlockSpec(shape, lambda i: (0,) * len(shape))
    grid = (n // BT,)
    out = pl.pallas_call(
        _rmlp_block,
        grid=grid,
        in_specs=[
            pl.BlockSpec((BT, IN_FEATURES), lambda i: (i, 0)),
            full((IN_FEATURES, HIDDEN)),
            full((HIDDEN,)),
            full((HIDDEN, NUM_EXPERTS)),
            full((NUM_EXPERTS,)),
            full((NUM_EXPERTS * HIDDEN, HIDDEN)),
            full((NUM_EXPERTS, HIDDEN)),
            full((HIDDEN, OUT_FEATURES)),
            full((OUT_FEATURES,)),
        ],
        out_specs=pl.BlockSpec((BT, OUT_FEATURES), lambda i: (i, 0)),
        out_shape=jax.ShapeDtypeStruct((n, OUT_FEATURES), jnp.float32),
    )(x, W_in, b_in, W_router, b_router, w_flat, b_experts, W_out, b_out)
    return out
